# Initial kernel scaffold; baseline (speedup 1.0000x reference)
import jax
import jax.numpy as jnp
from jax.experimental import pallas as pl


def kernel(xyz, params):
    raise NotImplementedError("write your pallas kernel here")


# speedup vs baseline: 6.3232x; 6.3232x over previous
import functools

import jax
import jax.numpy as jnp
from jax.experimental import pallas as pl


EPS = 1e-5


def _fps_kernel(xc_ref, out_ref, *, npoint, n):
    xc = xc_ref[...]
    b = xc.shape[0]
    iota = jax.lax.broadcasted_iota(jnp.int32, (b, n), 1)

    def body(i, carry):
        distance, far = carry
        onehot = jnp.where(iota == far, 1.0, 0.0)
        cent = jnp.sum(xc * onehot[:, None, :], axis=2)
        out_ref[pl.ds(i, 1)] = cent[None]
        diff = xc - cent[:, :, None]
        dist = jnp.sum(diff * diff, axis=1)
        distance = jnp.minimum(distance, dist)
        dmax = jnp.max(distance, axis=1, keepdims=True)
        far = jnp.min(jnp.where(distance == dmax, iota, n), axis=1,
                      keepdims=True)
        return distance, far

    distance0 = jnp.full((b, n), 1e10, jnp.float32)
    far0 = jnp.zeros((b, 1), jnp.int32)
    jax.lax.fori_loop(0, npoint, body, (distance0, far0))


def _fps(xc, npoint):
    b, _, n = xc.shape
    out = pl.pallas_call(
        functools.partial(_fps_kernel, npoint=npoint, n=n),
        out_shape=jax.ShapeDtypeStruct((npoint, b, 3), jnp.float32),
    )(xc)
    return jnp.transpose(out, (1, 0, 2))


def _bq_kernel(q_ref, xct_ref, table_ref, out_ref, *, r2, k, n):
    q = q_ref[0]
    x3 = xct_ref[0]
    table = table_ref[0]
    st = q.shape[0]
    d_feat = table.shape[1]

    qx = jnp.dot(q.astype(jnp.bfloat16), x3.astype(jnp.bfloat16),
                 preferred_element_type=jnp.float32)
    sq_q = jnp.sum(q * q, axis=1, keepdims=True)
    sq_x = jnp.sum(x3 * x3, axis=0, keepdims=True)
    d = (-2.0 * qx + sq_q) + sq_x
    iota = jax.lax.broadcasted_iota(jnp.int32, (st, n), 1)
    midx = jnp.where(d <= jnp.float32(r2), iota, n)

    qpad = jnp.concatenate(
        [q, jnp.zeros((st, d_feat - 3), jnp.float32)], axis=1)

    rows = []
    h0 = None
    for i in range(k):
        cur = jnp.min(midx, axis=1, keepdims=True)
        hit = midx == cur
        hf = jnp.where(hit, 1.0, 0.0)
        if i == 0:
            first = jnp.minimum(cur, n - 1)
            h0 = jnp.where(iota == first, 1.0, 0.0)
            h = h0
        else:
            vf = jnp.where(cur < n, 1.0, 0.0)
            h = hf * vf + h0 * (1.0 - vf)
        row = jnp.dot(h, table, preferred_element_type=jnp.float32,
                  precision=jax.lax.Precision.HIGHEST)
        rows.append(row - qpad)
        midx = jnp.where(hit, n, midx)

    out_ref[...] = jnp.stack(rows, axis=1)[None]


def _bq(new_xyz, xct, table, r2, k, s_tile):
    b, s, _ = new_xyz.shape
    n = xct.shape[2]
    d_feat = table.shape[2]
    grid = (b, s // s_tile)
    return pl.pallas_call(
        functools.partial(_bq_kernel, r2=r2, k=k, n=n),
        grid=grid,
        in_specs=[
            pl.BlockSpec((1, s_tile, 3), lambda i, j: (i, j, 0)),
            pl.BlockSpec((1, 3, n), lambda i, j: (i, 0, 0)),
            pl.BlockSpec((1, n, d_feat), lambda i, j: (i, 0, 0)),
        ],
        out_specs=pl.BlockSpec((1, s_tile, k, d_feat),
                               lambda i, j: (i, j, 0, 0)),
        out_shape=jax.ShapeDtypeStruct((b, s, k, d_feat), jnp.float32),
    )(new_xyz, xct, table)


def _stage_kernel(x_ref, w_ref, a_ref, c_ref, z_ref, s1_ref, s2_ref, *,
                  prologue):
    x = x_ref[...]
    if prologue:
        x = jnp.maximum(x * a_ref[...] + c_ref[...], 0.0)
    z = jnp.dot(x.astype(jnp.bfloat16), w_ref[...].astype(jnp.bfloat16),
                preferred_element_type=jnp.float32)
    z_ref[...] = z
    ps1 = jnp.sum(z, axis=0, keepdims=True)
    ps2 = jnp.sum(z * z, axis=0, keepdims=True)

    @pl.when(pl.program_id(0) == 0)
    def _init():
        s1_ref[...] = ps1
        s2_ref[...] = ps2

    @pl.when(pl.program_id(0) != 0)
    def _acc():
        s1_ref[...] += ps1
        s2_ref[...] += ps2


def _stage(x, wt, a, c, prologue):
    p, cin = x.shape
    cout = wt.shape[1]
    pt = min(p, 2048)
    grid = (p // pt,)
    z, s1, s2 = pl.pallas_call(
        functools.partial(_stage_kernel, prologue=prologue),
        grid=grid,
        in_specs=[
            pl.BlockSpec((pt, cin), lambda i: (i, 0)),
            pl.BlockSpec((cin, cout), lambda i: (0, 0)),
            pl.BlockSpec((1, cin), lambda i: (0, 0)),
            pl.BlockSpec((1, cin), lambda i: (0, 0)),
        ],
        out_specs=[
            pl.BlockSpec((pt, cout), lambda i: (i, 0)),
            pl.BlockSpec((1, cout), lambda i: (0, 0)),
            pl.BlockSpec((1, cout), lambda i: (0, 0)),
        ],
        out_shape=[
            jax.ShapeDtypeStruct((p, cout), jnp.float32),
            jax.ShapeDtypeStruct((1, cout), jnp.float32),
            jax.ShapeDtypeStruct((1, cout), jnp.float32),
        ],
    )(x, wt, a, c)
    return z, s1, s2


def _maxk_kernel(z_ref, a_ref, c_ref, out_ref):
    z = z_ref[...]
    y = jnp.maximum(z * a_ref[...] + c_ref[...], 0.0)
    out_ref[...] = jnp.max(y, axis=1)


def _maxk(z, a, c):
    g, k, cc = z.shape
    gt = max(1, min(g, (1 << 22) // (k * cc * 4)))
    while g % gt:
        gt -= 1
    grid = (g // gt,)
    return pl.pallas_call(
        _maxk_kernel,
        grid=grid,
        in_specs=[
            pl.BlockSpec((gt, k, cc), lambda i: (i, 0, 0)),
            pl.BlockSpec((1, 1, cc), lambda i: (0, 0, 0)),
            pl.BlockSpec((1, 1, cc), lambda i: (0, 0, 0)),
        ],
        out_specs=pl.BlockSpec((gt, cc), lambda i: (i, 0)),
        out_shape=jax.ShapeDtypeStruct((g, cc), jnp.float32),
    )(z, a, c)


def _bn_affine(s1, s2, p, g, beta):
    mu = s1 / p
    var = s2 / p - mu * mu
    a = g[None, :] / jnp.sqrt(var + EPS)
    c = beta[None, :] - mu * a
    return a, c


def _run_mlp(x2d, layer_params, p):
    a = jnp.ones((1, x2d.shape[1]), jnp.float32)
    c = jnp.zeros((1, x2d.shape[1]), jnp.float32)
    for i, (w, _b, g, beta) in enumerate(layer_params):
        z, s1, s2 = _stage(x2d, jnp.transpose(w), a, c, prologue=(i > 0))
        a, c = _bn_affine(s1, s2, p, g, beta)
        x2d = z
    return x2d, a, c


def _sa_layer(xc, feats_t, npoint, r2, k, layer_params, s_tile):
    b, _, n = xc.shape
    new_xyz = _fps(xc, npoint)
    table = jnp.concatenate([jnp.transpose(xc, (0, 2, 1)), feats_t], axis=2)
    grouped = _bq(new_xyz, xc, table, r2, k, s_tile)
    d_feat = grouped.shape[3]
    p = b * npoint * k
    z, a, c = _run_mlp(grouped.reshape(p, d_feat), layer_params, p)
    cout = z.shape[1]
    pooled = _maxk(z.reshape(b * npoint, k, cout), a[None], c[None])
    return new_xyz, pooled.reshape(b, npoint, cout)


def kernel(xyz, params):
    xyz = xyz.astype(jnp.float32)
    b = xyz.shape[0]
    coords = xyz[:, :3, :]
    norm_t = jnp.transpose(xyz[:, 3:, :], (0, 2, 1))

    l1_xyz, l1_p = _sa_layer(coords, norm_t, 1024, 0.1 ** 2, 16,
                             params[0], 128)
    l2_xyz, l2_p = _sa_layer(jnp.transpose(l1_xyz, (0, 2, 1)), l1_p,
                             512, 0.2 ** 2, 32, params[1], 128)
    l3_xyz, l3_p = _sa_layer(jnp.transpose(l2_xyz, (0, 2, 1)), l2_p,
                             128, 0.4 ** 2, 64, params[2], 128)

    x4 = jnp.concatenate([l3_xyz, l3_p], axis=2)
    n4 = x4.shape[1]
    p4 = b * n4
    z, a, c = _run_mlp(x4.reshape(p4, x4.shape[2]), params[3], p4)
    out = _maxk(z.reshape(b, n4, z.shape[1]), a[None], c[None])
    return out.reshape(b, z.shape[1])


# speedup vs baseline: 10.3987x; 1.6445x over previous
import functools

import jax
import jax.numpy as jnp
from jax import lax
from jax.experimental import pallas as pl
from jax.experimental.pallas import tpu as pltpu
from jax.experimental.pallas import tpu_sc as plsc


EPS = 1e-5


def _fps_kernel(xc_ref, out_ref, *, npoint, n):
    xc = xc_ref[...]
    b = xc.shape[0]
    iota = jax.lax.broadcasted_iota(jnp.int32, (b, n), 1)

    def body(i, carry):
        distance, far = carry
        onehot = jnp.where(iota == far, 1.0, 0.0)
        cent = jnp.sum(xc * onehot[:, None, :], axis=2)
        out_ref[pl.ds(i, 1)] = cent[None]
        diff = xc - cent[:, :, None]
        dist = jnp.sum(diff * diff, axis=1)
        distance = jnp.minimum(distance, dist)
        dmax = jnp.max(distance, axis=1, keepdims=True)
        far = jnp.min(jnp.where(distance == dmax, iota, n), axis=1,
                      keepdims=True)
        return distance, far

    distance0 = jnp.full((b, n), 1e10, jnp.float32)
    far0 = jnp.zeros((b, 1), jnp.int32)
    jax.lax.fori_loop(0, npoint, body, (distance0, far0))


def _fps(xc, npoint):
    b, _, n = xc.shape
    out = pl.pallas_call(
        functools.partial(_fps_kernel, npoint=npoint, n=n),
        out_shape=jax.ShapeDtypeStruct((npoint, b, 3), jnp.float32),
    )(xc)
    return jnp.transpose(out, (1, 0, 2))


def _bq_kernel(q_ref, xct_ref, table_ref, out_ref, *, r2, k, n):
    q = q_ref[0]
    x3 = xct_ref[0]
    table = table_ref[0]
    st = q.shape[0]
    d_feat = table.shape[1]

    qx = jnp.dot(q.astype(jnp.bfloat16), x3.astype(jnp.bfloat16),
                 preferred_element_type=jnp.float32)
    sq_q = jnp.sum(q * q, axis=1, keepdims=True)
    sq_x = jnp.sum(x3 * x3, axis=0, keepdims=True)
    d = (-2.0 * qx + sq_q) + sq_x
    iota = jax.lax.broadcasted_iota(jnp.int32, (st, n), 1)
    midx = jnp.where(d <= jnp.float32(r2), iota, n)

    qpad = jnp.concatenate(
        [q, jnp.zeros((st, d_feat - 3), jnp.float32)], axis=1)

    rows = []
    h0 = None
    for i in range(k):
        cur = jnp.min(midx, axis=1, keepdims=True)
        hit = midx == cur
        hf = jnp.where(hit, 1.0, 0.0)
        if i == 0:
            first = jnp.minimum(cur, n - 1)
            h0 = jnp.where(iota == first, 1.0, 0.0)
            h = h0
        else:
            vf = jnp.where(cur < n, 1.0, 0.0)
            h = hf * vf + h0 * (1.0 - vf)
        row = jnp.dot(h, table, preferred_element_type=jnp.float32,
                  precision=jax.lax.Precision.HIGHEST)
        rows.append(row - qpad)
        midx = jnp.where(hit, n, midx)

    out_ref[...] = jnp.stack(rows, axis=1)[None]


def _bq(new_xyz, xct, table, r2, k, s_tile):
    b, s, _ = new_xyz.shape
    n = xct.shape[2]
    d_feat = table.shape[2]
    grid = (b, s // s_tile)
    return pl.pallas_call(
        functools.partial(_bq_kernel, r2=r2, k=k, n=n),
        grid=grid,
        in_specs=[
            pl.BlockSpec((1, s_tile, 3), lambda i, j: (i, j, 0)),
            pl.BlockSpec((1, 3, n), lambda i, j: (i, 0, 0)),
            pl.BlockSpec((1, n, d_feat), lambda i, j: (i, 0, 0)),
        ],
        out_specs=pl.BlockSpec((1, s_tile, k, d_feat),
                               lambda i, j: (i, j, 0, 0)),
        out_shape=jax.ShapeDtypeStruct((b, s, k, d_feat), jnp.float32),
    )(new_xyz, xct, table)


def _bq_idx_kernel(q_ref, xct_ref, out_ref, *, r2, k, n):
    q = q_ref[0]
    x3 = xct_ref[0]
    st = q.shape[0]
    qx = jnp.dot(q.astype(jnp.bfloat16), x3.astype(jnp.bfloat16),
                 preferred_element_type=jnp.float32)
    sq_q = jnp.sum(q * q, axis=1, keepdims=True)
    sq_x = jnp.sum(x3 * x3, axis=0, keepdims=True)
    d = (-2.0 * qx + sq_q) + sq_x
    iota = jax.lax.broadcasted_iota(jnp.int32, (st, n), 1)
    midx = jnp.where(d <= jnp.float32(r2), iota, n)

    cols = []
    first = None
    for i in range(k):
        cur = jnp.min(midx, axis=1, keepdims=True)
        if i == 0:
            first = jnp.minimum(cur, n - 1)
            cols.append(first)
        else:
            cols.append(jnp.where(cur < n, cur, first))
        midx = jnp.where(midx == cur, n, midx)
    out_ref[...] = jnp.concatenate(cols, axis=1)[None]


def _bq_idx(new_xyz, xct, r2, k, s_tile):
    b, s, _ = new_xyz.shape
    n = xct.shape[2]
    grid = (b, s // s_tile)
    return pl.pallas_call(
        functools.partial(_bq_idx_kernel, r2=r2, k=k, n=n),
        grid=grid,
        in_specs=[
            pl.BlockSpec((1, s_tile, 3), lambda i, j: (i, j, 0)),
            pl.BlockSpec((1, 3, n), lambda i, j: (i, 0, 0)),
        ],
        out_specs=pl.BlockSpec((1, s_tile, k), lambda i, j: (i, j, 0)),
        out_shape=jax.ShapeDtypeStruct((b, s, k), jnp.int32),
    )(new_xyz, xct)


_SC_CH = 128


def _sc_gather(table, idx2d):
    n_ch, ch = idx2d.shape
    d = table.shape[1]
    p = n_ch * ch
    info = plsc.get_sparse_core_info()
    nw = info.num_cores * info.num_subcores
    ch_per_w = n_ch // nw
    mesh = plsc.VectorSubcoreMesh(core_axis_name="c", subcore_axis_name="s")

    @functools.partial(
        pl.kernel, mesh=mesh,
        out_type=jax.ShapeDtypeStruct((p, d), jnp.float32),
        scratch_types=[
            pltpu.VMEM((ch_per_w, ch), jnp.int32),
            pltpu.VMEM((ch, d), jnp.float32),
            pltpu.SemaphoreType.DMA,
        ],
    )
    def k(table_hbm, idx_hbm, out_hbm, idxv, buf, sem):
        wid = lax.axis_index("s") * info.num_cores + lax.axis_index("c")
        base = wid * ch_per_w
        pltpu.sync_copy(idx_hbm.at[pl.ds(base, ch_per_w)], idxv)

        def body(j, carry):
            pltpu.async_copy(table_hbm.at[idxv.at[j]], buf, sem).wait()
            pltpu.sync_copy(buf, out_hbm.at[pl.ds((base + j) * ch, ch)])
            return carry

        lax.fori_loop(0, ch_per_w, body, 0)

    return k(table, idx2d)


def _pad16(x):
    c = x.shape[-1]
    pad = (-c) % 128
    if pad:
        x = jnp.concatenate(
            [x, jnp.zeros(x.shape[:-1] + (pad,), x.dtype)], axis=-1)
    return x


def _stage1g_kernel(x_ref, q_ref, w_ref, z_ref, s1_ref, s2_ref):
    x = x_ref[...] - q_ref[...][:, None, :]
    gt, kk, dd = x.shape
    x = x.reshape(gt * kk, dd)
    z = jnp.dot(x.astype(jnp.bfloat16), w_ref[...].astype(jnp.bfloat16),
                preferred_element_type=jnp.float32)
    z_ref[...] = z
    ps1 = jnp.sum(z, axis=0, keepdims=True)
    ps2 = jnp.sum(z * z, axis=0, keepdims=True)

    @pl.when(pl.program_id(0) == 0)
    def _init():
        s1_ref[...] = ps1
        s2_ref[...] = ps2

    @pl.when(pl.program_id(0) != 0)
    def _acc():
        s1_ref[...] += ps1
        s2_ref[...] += ps2


def _stage1g(rows3d, qpad, wt):
    g, k, d16 = rows3d.shape
    cout = wt.shape[1]
    p = g * k
    pt = min(p, 2048)
    gt = pt // k
    grid = (g // gt,)
    z, s1, s2 = pl.pallas_call(
        _stage1g_kernel,
        grid=grid,
        in_specs=[
            pl.BlockSpec((gt, k, d16), lambda i: (i, 0, 0)),
            pl.BlockSpec((gt, d16), lambda i: (i, 0)),
            pl.BlockSpec((d16, cout), lambda i: (0, 0)),
        ],
        out_specs=[
            pl.BlockSpec((pt, cout), lambda i: (i, 0)),
            pl.BlockSpec((1, cout), lambda i: (0, 0)),
            pl.BlockSpec((1, cout), lambda i: (0, 0)),
        ],
        out_shape=[
            jax.ShapeDtypeStruct((p, cout), jnp.float32),
            jax.ShapeDtypeStruct((1, cout), jnp.float32),
            jax.ShapeDtypeStruct((1, cout), jnp.float32),
        ],
    )(rows3d, qpad, wt)
    return z, s1, s2


def _stage_kernel(x_ref, w_ref, a_ref, c_ref, z_ref, s1_ref, s2_ref, *,
                  prologue):
    x = x_ref[...]
    if prologue:
        x = jnp.maximum(x * a_ref[...] + c_ref[...], 0.0)
    z = jnp.dot(x.astype(jnp.bfloat16), w_ref[...].astype(jnp.bfloat16),
                preferred_element_type=jnp.float32)
    z_ref[...] = z
    ps1 = jnp.sum(z, axis=0, keepdims=True)
    ps2 = jnp.sum(z * z, axis=0, keepdims=True)

    @pl.when(pl.program_id(0) == 0)
    def _init():
        s1_ref[...] = ps1
        s2_ref[...] = ps2

    @pl.when(pl.program_id(0) != 0)
    def _acc():
        s1_ref[...] += ps1
        s2_ref[...] += ps2


def _stage(x, wt, a, c, prologue):
    p, cin = x.shape
    cout = wt.shape[1]
    pt = min(p, 2048)
    grid = (p // pt,)
    z, s1, s2 = pl.pallas_call(
        functools.partial(_stage_kernel, prologue=prologue),
        grid=grid,
        in_specs=[
            pl.BlockSpec((pt, cin), lambda i: (i, 0)),
            pl.BlockSpec((cin, cout), lambda i: (0, 0)),
            pl.BlockSpec((1, cin), lambda i: (0, 0)),
            pl.BlockSpec((1, cin), lambda i: (0, 0)),
        ],
        out_specs=[
            pl.BlockSpec((pt, cout), lambda i: (i, 0)),
            pl.BlockSpec((1, cout), lambda i: (0, 0)),
            pl.BlockSpec((1, cout), lambda i: (0, 0)),
        ],
        out_shape=[
            jax.ShapeDtypeStruct((p, cout), jnp.float32),
            jax.ShapeDtypeStruct((1, cout), jnp.float32),
            jax.ShapeDtypeStruct((1, cout), jnp.float32),
        ],
    )(x, wt, a, c)
    return z, s1, s2


def _maxk_kernel(z_ref, a_ref, c_ref, out_ref):
    z = z_ref[...]
    y = jnp.maximum(z * a_ref[...] + c_ref[...], 0.0)
    out_ref[...] = jnp.max(y, axis=1)


def _maxk(z, a, c):
    g, k, cc = z.shape
    gt = max(1, min(g, (1 << 22) // (k * cc * 4)))
    while g % gt:
        gt -= 1
    grid = (g // gt,)
    return pl.pallas_call(
        _maxk_kernel,
        grid=grid,
        in_specs=[
            pl.BlockSpec((gt, k, cc), lambda i: (i, 0, 0)),
            pl.BlockSpec((1, 1, cc), lambda i: (0, 0, 0)),
            pl.BlockSpec((1, 1, cc), lambda i: (0, 0, 0)),
        ],
        out_specs=pl.BlockSpec((gt, cc), lambda i: (i, 0)),
        out_shape=jax.ShapeDtypeStruct((g, cc), jnp.float32),
    )(z, a, c)


def _bn_affine(s1, s2, p, g, beta):
    mu = s1 / p
    var = s2 / p - mu * mu
    a = g[None, :] / jnp.sqrt(var + EPS)
    c = beta[None, :] - mu * a
    return a, c


def _run_mlp(x2d, layer_params, p):
    a = jnp.ones((1, x2d.shape[1]), jnp.float32)
    c = jnp.zeros((1, x2d.shape[1]), jnp.float32)
    for i, (w, _b, g, beta) in enumerate(layer_params):
        z, s1, s2 = _stage(x2d, jnp.transpose(w), a, c, prologue=(i > 0))
        a, c = _bn_affine(s1, s2, p, g, beta)
        x2d = z
    return x2d, a, c


def _sa_layer(xc, feats_t, npoint, r2, k, layer_params, s_tile):
    b, _, n = xc.shape
    new_xyz = _fps(xc, npoint)
    table = _pad16(
        jnp.concatenate([jnp.transpose(xc, (0, 2, 1)), feats_t], axis=2))
    d16 = table.shape[2]
    idx = _bq_idx(new_xyz, xc, r2, k, s_tile)
    flat_idx = idx + (jnp.arange(b, dtype=jnp.int32) * n)[:, None, None]
    p = b * npoint * k
    rows = _sc_gather(table.reshape(b * n, d16),
                      flat_idx.reshape(p // _SC_CH, _SC_CH))

    g_rows = b * npoint
    qpad = jnp.concatenate(
        [new_xyz.reshape(g_rows, 3),
         jnp.zeros((g_rows, d16 - 3), jnp.float32)], axis=1)
    w0, _b0, g0, beta0 = layer_params[0]
    wt0 = jnp.transpose(w0)
    wt0 = jnp.concatenate(
        [wt0, jnp.zeros((d16 - wt0.shape[0], wt0.shape[1]), jnp.float32)],
        axis=0)
    z, s1, s2 = _stage1g(rows.reshape(g_rows, k, d16), qpad, wt0)
    a, c = _bn_affine(s1, s2, p, g0, beta0)
    for (w, _bb, g, beta) in layer_params[1:]:
        z, s1, s2 = _stage(z, jnp.transpose(w), a, c, prologue=True)
        a, c = _bn_affine(s1, s2, p, g, beta)
    cout = z.shape[1]
    pooled = _maxk(z.reshape(g_rows, k, cout), a[None], c[None])
    return new_xyz, pooled.reshape(b, npoint, cout)


def kernel(xyz, params):
    xyz = xyz.astype(jnp.float32)
    b = xyz.shape[0]
    coords = xyz[:, :3, :]
    norm_t = jnp.transpose(xyz[:, 3:, :], (0, 2, 1))

    l1_xyz, l1_p = _sa_layer(coords, norm_t, 1024, 0.1 ** 2, 16,
                             params[0], 128)
    l2_xyz, l2_p = _sa_layer(jnp.transpose(l1_xyz, (0, 2, 1)), l1_p,
                             512, 0.2 ** 2, 32, params[1], 128)
    l3_xyz, l3_p = _sa_layer(jnp.transpose(l2_xyz, (0, 2, 1)), l2_p,
                             128, 0.4 ** 2, 64, params[2], 128)

    x4 = jnp.concatenate([l3_xyz, l3_p], axis=2)
    n4 = x4.shape[1]
    p4 = b * n4
    z, a, c = _run_mlp(x4.reshape(p4, x4.shape[2]), params[3], p4)
    out = _maxk(z.reshape(b, n4, z.shape[1]), a[None], c[None])
    return out.reshape(b, z.shape[1])


# speedup vs baseline: 14.0048x; 1.3468x over previous
import functools

import jax
import jax.numpy as jnp
from jax import lax
from jax.experimental import pallas as pl
from jax.experimental.pallas import tpu as pltpu
from jax.experimental.pallas import tpu_sc as plsc


EPS = 1e-5


def _fps_kernel(xc_ref, out_ref, *, npoint, n):
    x0 = xc_ref[:, 0, :]
    x1 = xc_ref[:, 1, :]
    x2 = xc_ref[:, 2, :]
    b = x0.shape[0]
    iota = jax.lax.broadcasted_iota(jnp.int32, (b, n), 1)

    def body(i, carry):
        distance, far = carry
        onehot = jnp.where(iota == far, 1.0, 0.0)
        cx = jnp.sum(x0 * onehot, axis=1, keepdims=True)
        cy = jnp.sum(x1 * onehot, axis=1, keepdims=True)
        cz = jnp.sum(x2 * onehot, axis=1, keepdims=True)
        out_ref[pl.ds(i, 1)] = jnp.concatenate([cx, cy, cz], axis=1)[None]
        d0 = x0 - cx
        d1 = x1 - cy
        d2 = x2 - cz
        dist = (d0 * d0 + d1 * d1) + d2 * d2
        distance = jnp.minimum(distance, dist)
        dmax = jnp.max(distance, axis=1, keepdims=True)
        far = jnp.min(jnp.where(distance == dmax, iota, n), axis=1,
                      keepdims=True)
        return distance, far

    distance0 = jnp.full((b, n), 1e10, jnp.float32)
    far0 = jnp.zeros((b, 1), jnp.int32)
    jax.lax.fori_loop(0, npoint, body, (distance0, far0))


def _fps(xc, npoint):
    b, _, n = xc.shape
    out = pl.pallas_call(
        functools.partial(_fps_kernel, npoint=npoint, n=n),
        out_shape=jax.ShapeDtypeStruct((npoint, b, 3), jnp.float32),
    )(xc)
    return jnp.transpose(out, (1, 0, 2))


def _bq_kernel(q_ref, xct_ref, table_ref, out_ref, *, r2, k, n):
    q = q_ref[0]
    x3 = xct_ref[0]
    table = table_ref[0]
    st = q.shape[0]
    d_feat = table.shape[1]

    qx = jnp.dot(q.astype(jnp.bfloat16), x3.astype(jnp.bfloat16),
                 preferred_element_type=jnp.float32)
    sq_q = jnp.sum(q * q, axis=1, keepdims=True)
    sq_x = jnp.sum(x3 * x3, axis=0, keepdims=True)
    d = (-2.0 * qx + sq_q) + sq_x
    iota = jax.lax.broadcasted_iota(jnp.int32, (st, n), 1)
    midx = jnp.where(d <= jnp.float32(r2), iota, n)

    qpad = jnp.concatenate(
        [q, jnp.zeros((st, d_feat - 3), jnp.float32)], axis=1)

    rows = []
    h0 = None
    for i in range(k):
        cur = jnp.min(midx, axis=1, keepdims=True)
        hit = midx == cur
        hf = jnp.where(hit, 1.0, 0.0)
        if i == 0:
            first = jnp.minimum(cur, n - 1)
            h0 = jnp.where(iota == first, 1.0, 0.0)
            h = h0
        else:
            vf = jnp.where(cur < n, 1.0, 0.0)
            h = hf * vf + h0 * (1.0 - vf)
        row = jnp.dot(h, table, preferred_element_type=jnp.float32,
                  precision=jax.lax.Precision.HIGHEST)
        rows.append(row - qpad)
        midx = jnp.where(hit, n, midx)

    out_ref[...] = jnp.stack(rows, axis=1)[None]


def _bq(new_xyz, xct, table, r2, k, s_tile):
    b, s, _ = new_xyz.shape
    n = xct.shape[2]
    d_feat = table.shape[2]
    grid = (b, s // s_tile)
    return pl.pallas_call(
        functools.partial(_bq_kernel, r2=r2, k=k, n=n),
        grid=grid,
        in_specs=[
            pl.BlockSpec((1, s_tile, 3), lambda i, j: (i, j, 0)),
            pl.BlockSpec((1, 3, n), lambda i, j: (i, 0, 0)),
            pl.BlockSpec((1, n, d_feat), lambda i, j: (i, 0, 0)),
        ],
        out_specs=pl.BlockSpec((1, s_tile, k, d_feat),
                               lambda i, j: (i, j, 0, 0)),
        out_shape=jax.ShapeDtypeStruct((b, s, k, d_feat), jnp.float32),
    )(new_xyz, xct, table)


def _bq_idx_kernel(q_ref, xct_ref, out_ref, *, r2, k, n):
    q = q_ref[0]
    x3 = xct_ref[0]
    st = q.shape[0]
    qx = jnp.dot(q.astype(jnp.bfloat16), x3.astype(jnp.bfloat16),
                 preferred_element_type=jnp.float32)
    sq_q = jnp.sum(q * q, axis=1, keepdims=True)
    sq_x = jnp.sum(x3 * x3, axis=0, keepdims=True)
    d = (-2.0 * qx + sq_q) + sq_x
    iota = jax.lax.broadcasted_iota(jnp.int32, (st, n), 1)
    midx = jnp.where(d <= jnp.float32(r2), iota, n)

    cols = []
    first = None
    for i in range(k):
        cur = jnp.min(midx, axis=1, keepdims=True)
        if i == 0:
            first = jnp.minimum(cur, n - 1)
            cols.append(first)
        else:
            cols.append(jnp.where(cur < n, cur, first))
        midx = jnp.where(midx == cur, n, midx)
    out_ref[...] = jnp.concatenate(cols, axis=1)[None]


def _bq_idx(new_xyz, xct, r2, k, s_tile):
    b, s, _ = new_xyz.shape
    n = xct.shape[2]
    grid = (b, s // s_tile)
    return pl.pallas_call(
        functools.partial(_bq_idx_kernel, r2=r2, k=k, n=n),
        grid=grid,
        in_specs=[
            pl.BlockSpec((1, s_tile, 3), lambda i, j: (i, j, 0)),
            pl.BlockSpec((1, 3, n), lambda i, j: (i, 0, 0)),
        ],
        out_specs=pl.BlockSpec((1, s_tile, k), lambda i, j: (i, j, 0)),
        out_shape=jax.ShapeDtypeStruct((b, s, k), jnp.int32),
    )(new_xyz, xct)


_SC_CH = 128


def _sc_gather(table, idx2d):
    n_ch, ch = idx2d.shape
    d = table.shape[1]
    p = n_ch * ch
    info = plsc.get_sparse_core_info()
    nw = info.num_cores * info.num_subcores
    ch_per_w = n_ch // nw
    mesh = plsc.VectorSubcoreMesh(core_axis_name="c", subcore_axis_name="s")

    @functools.partial(
        pl.kernel, mesh=mesh,
        out_type=jax.ShapeDtypeStruct((p, d), jnp.float32),
        scratch_types=[
            pltpu.VMEM((ch_per_w, ch), jnp.int32),
            pltpu.VMEM((ch, d), jnp.float32),
            pltpu.SemaphoreType.DMA,
        ],
    )
    def k(table_hbm, idx_hbm, out_hbm, idxv, buf, sem):
        wid = lax.axis_index("s") * info.num_cores + lax.axis_index("c")
        base = wid * ch_per_w
        pltpu.sync_copy(idx_hbm.at[pl.ds(base, ch_per_w)], idxv)

        def body(j, carry):
            pltpu.async_copy(table_hbm.at[idxv.at[j]], buf, sem).wait()
            pltpu.sync_copy(buf, out_hbm.at[pl.ds((base + j) * ch, ch)])
            return carry

        lax.fori_loop(0, ch_per_w, body, 0)

    return k(table, idx2d)


def _pad16(x):
    c = x.shape[-1]
    pad = (-c) % 128
    if pad:
        x = jnp.concatenate(
            [x, jnp.zeros(x.shape[:-1] + (pad,), x.dtype)], axis=-1)
    return x


def _stage1g_kernel(x_ref, q_ref, w_ref, z_ref, s1_ref, s2_ref):
    x = x_ref[...] - q_ref[...][:, None, :]
    gt, kk, dd = x.shape
    x = x.reshape(gt * kk, dd)
    z = jnp.dot(x.astype(jnp.bfloat16), w_ref[...].astype(jnp.bfloat16),
                preferred_element_type=jnp.float32)
    z_ref[...] = z
    ps1 = jnp.sum(z, axis=0, keepdims=True)
    ps2 = jnp.sum(z * z, axis=0, keepdims=True)

    @pl.when(pl.program_id(0) == 0)
    def _init():
        s1_ref[...] = ps1
        s2_ref[...] = ps2

    @pl.when(pl.program_id(0) != 0)
    def _acc():
        s1_ref[...] += ps1
        s2_ref[...] += ps2


def _stage1g(rows3d, qpad, wt):
    g, k, d16 = rows3d.shape
    cout = wt.shape[1]
    p = g * k
    pt = min(p, 2048)
    gt = pt // k
    grid = (g // gt,)
    z, s1, s2 = pl.pallas_call(
        _stage1g_kernel,
        grid=grid,
        in_specs=[
            pl.BlockSpec((gt, k, d16), lambda i: (i, 0, 0)),
            pl.BlockSpec((gt, d16), lambda i: (i, 0)),
            pl.BlockSpec((d16, cout), lambda i: (0, 0)),
        ],
        out_specs=[
            pl.BlockSpec((pt, cout), lambda i: (i, 0)),
            pl.BlockSpec((1, cout), lambda i: (0, 0)),
            pl.BlockSpec((1, cout), lambda i: (0, 0)),
        ],
        out_shape=[
            jax.ShapeDtypeStruct((p, cout), jnp.float32),
            jax.ShapeDtypeStruct((1, cout), jnp.float32),
            jax.ShapeDtypeStruct((1, cout), jnp.float32),
        ],
    )(rows3d, qpad, wt)
    return z, s1, s2


def _stage_kernel(x_ref, w_ref, a_ref, c_ref, z_ref, s1_ref, s2_ref, *,
                  prologue):
    x = x_ref[...]
    if prologue:
        x = jnp.maximum(x * a_ref[...] + c_ref[...], 0.0)
    z = jnp.dot(x.astype(jnp.bfloat16), w_ref[...].astype(jnp.bfloat16),
                preferred_element_type=jnp.float32)
    z_ref[...] = z
    ps1 = jnp.sum(z, axis=0, keepdims=True)
    ps2 = jnp.sum(z * z, axis=0, keepdims=True)

    @pl.when(pl.program_id(0) == 0)
    def _init():
        s1_ref[...] = ps1
        s2_ref[...] = ps2

    @pl.when(pl.program_id(0) != 0)
    def _acc():
        s1_ref[...] += ps1
        s2_ref[...] += ps2


def _stage(x, wt, a, c, prologue):
    p, cin = x.shape
    cout = wt.shape[1]
    pt = min(p, 2048)
    grid = (p // pt,)
    z, s1, s2 = pl.pallas_call(
        functools.partial(_stage_kernel, prologue=prologue),
        grid=grid,
        in_specs=[
            pl.BlockSpec((pt, cin), lambda i: (i, 0)),
            pl.BlockSpec((cin, cout), lambda i: (0, 0)),
            pl.BlockSpec((1, cin), lambda i: (0, 0)),
            pl.BlockSpec((1, cin), lambda i: (0, 0)),
        ],
        out_specs=[
            pl.BlockSpec((pt, cout), lambda i: (i, 0)),
            pl.BlockSpec((1, cout), lambda i: (0, 0)),
            pl.BlockSpec((1, cout), lambda i: (0, 0)),
        ],
        out_shape=[
            jax.ShapeDtypeStruct((p, cout), jnp.float32),
            jax.ShapeDtypeStruct((1, cout), jnp.float32),
            jax.ShapeDtypeStruct((1, cout), jnp.float32),
        ],
    )(x, wt, a, c)
    return z, s1, s2


def _maxk_kernel(z_ref, a_ref, c_ref, out_ref):
    z = z_ref[...]
    y = jnp.maximum(z * a_ref[...] + c_ref[...], 0.0)
    out_ref[...] = jnp.max(y, axis=1)


def _maxk(z, a, c):
    g, k, cc = z.shape
    gt = max(1, min(g, (1 << 22) // (k * cc * 4)))
    while g % gt:
        gt -= 1
    grid = (g // gt,)
    return pl.pallas_call(
        _maxk_kernel,
        grid=grid,
        in_specs=[
            pl.BlockSpec((gt, k, cc), lambda i: (i, 0, 0)),
            pl.BlockSpec((1, 1, cc), lambda i: (0, 0, 0)),
            pl.BlockSpec((1, 1, cc), lambda i: (0, 0, 0)),
        ],
        out_specs=pl.BlockSpec((gt, cc), lambda i: (i, 0)),
        out_shape=jax.ShapeDtypeStruct((g, cc), jnp.float32),
    )(z, a, c)


def _bn_affine(s1, s2, p, g, beta):
    mu = s1 / p
    var = s2 / p - mu * mu
    a = g[None, :] / jnp.sqrt(var + EPS)
    c = beta[None, :] - mu * a
    return a, c


def _run_mlp(x2d, layer_params, p):
    a = jnp.ones((1, x2d.shape[1]), jnp.float32)
    c = jnp.zeros((1, x2d.shape[1]), jnp.float32)
    for i, (w, _b, g, beta) in enumerate(layer_params):
        z, s1, s2 = _stage(x2d, jnp.transpose(w), a, c, prologue=(i > 0))
        a, c = _bn_affine(s1, s2, p, g, beta)
        x2d = z
    return x2d, a, c


def _sa_layer(xc, feats_t, npoint, r2, k, layer_params, s_tile):
    b, _, n = xc.shape
    new_xyz = _fps(xc, npoint)
    table = _pad16(
        jnp.concatenate([jnp.transpose(xc, (0, 2, 1)), feats_t], axis=2))
    d16 = table.shape[2]
    idx = _bq_idx(new_xyz, xc, r2, k, s_tile)
    flat_idx = idx + (jnp.arange(b, dtype=jnp.int32) * n)[:, None, None]
    p = b * npoint * k
    rows = _sc_gather(table.reshape(b * n, d16),
                      flat_idx.reshape(p // _SC_CH, _SC_CH))

    g_rows = b * npoint
    qpad = jnp.concatenate(
        [new_xyz.reshape(g_rows, 3),
         jnp.zeros((g_rows, d16 - 3), jnp.float32)], axis=1)
    w0, _b0, g0, beta0 = layer_params[0]
    wt0 = jnp.transpose(w0)
    wt0 = jnp.concatenate(
        [wt0, jnp.zeros((d16 - wt0.shape[0], wt0.shape[1]), jnp.float32)],
        axis=0)
    z, s1, s2 = _stage1g(rows.reshape(g_rows, k, d16), qpad, wt0)
    a, c = _bn_affine(s1, s2, p, g0, beta0)
    for (w, _bb, g, beta) in layer_params[1:]:
        z, s1, s2 = _stage(z, jnp.transpose(w), a, c, prologue=True)
        a, c = _bn_affine(s1, s2, p, g, beta)
    cout = z.shape[1]
    pooled = _maxk(z.reshape(g_rows, k, cout), a[None], c[None])
    return new_xyz, pooled.reshape(b, npoint, cout)


def kernel(xyz, params):
    xyz = xyz.astype(jnp.float32)
    b = xyz.shape[0]
    coords = xyz[:, :3, :]
    norm_t = jnp.transpose(xyz[:, 3:, :], (0, 2, 1))

    l1_xyz, l1_p = _sa_layer(coords, norm_t, 1024, 0.1 ** 2, 16,
                             params[0], 128)
    l2_xyz, l2_p = _sa_layer(jnp.transpose(l1_xyz, (0, 2, 1)), l1_p,
                             512, 0.2 ** 2, 32, params[1], 128)
    l3_xyz, l3_p = _sa_layer(jnp.transpose(l2_xyz, (0, 2, 1)), l2_p,
                             128, 0.4 ** 2, 64, params[2], 128)

    x4 = jnp.concatenate([l3_xyz, l3_p], axis=2)
    n4 = x4.shape[1]
    p4 = b * n4
    z, a, c = _run_mlp(x4.reshape(p4, x4.shape[2]), params[3], p4)
    out = _maxk(z.reshape(b, n4, z.shape[1]), a[None], c[None])
    return out.reshape(b, z.shape[1])
